# baseline (speedup 1.0000x reference)
import jax
import jax.numpy as jnp
import numpy as np
from jax.experimental import pallas as pl

B, N, DIN, D, H, DH, L, K, DFF, DOUT = 4, 2048, 2, 128, 4, 32, 3, 16, 256, 128


def _ln(x, s, b):
    m = x.mean(-1, keepdims=True)
    v = x.var(-1, keepdims=True)
    return (x - m) / jnp.sqrt(v + 1e-5) * s + b


def _final_mm_kernel(x_ref, w_ref, o_ref):
    o_ref[...] = jnp.dot(x_ref[...], w_ref[...], preferred_element_type=jnp.float32)


def kernel(h, input_pos, Wproj, bproj, Wq, Wk, Wv, Wo, Wrel, ln1_s, ln1_b, W1, b1, W2, b2, ln2_s, ln2_b, Wout):
    pos = input_pos
    x = h @ Wproj + bproj
    sq = (pos ** 2).sum(-1)
    d2 = sq[:, :, None] + sq[:, None, :] - 2.0 * jnp.einsum('bnd,bmd->bnm', pos, pos)
    _, idx = jax.lax.top_k(-d2, K)
    gather = jax.vmap(lambda f, i: f[i])
    pos_n = gather(pos, idx)
    rel = pos_n - pos[:, :, None, :]
    for l in range(L):
        xn = _ln(x, ln1_s[l], ln1_b[l])
        q = (xn @ Wq[l]).reshape(B, N, H, DH)
        kf = xn @ Wk[l]
        vf = xn @ Wv[l]
        kn = gather(kf, idx).reshape(B, N, K, H, DH)
        vn = gather(vf, idx).reshape(B, N, K, H, DH)
        logits = jnp.einsum('bnhd,bnkhd->bnkh', q, kn) / np.sqrt(DH) + rel @ Wrel[l]
        attn = jax.nn.softmax(logits, axis=2)
        o = jnp.einsum('bnkh,bnkhd->bnhd', attn, vn).reshape(B, N, D) @ Wo[l]
        x = x + o
        xn2 = _ln(x, ln2_s[l], ln2_b[l])
        x = x + (jax.nn.gelu(xn2 @ W1[l] + b1[l]) @ W2[l] + b2[l])
    out = pl.pallas_call(
        _final_mm_kernel,
        out_shape=jax.ShapeDtypeStruct((B, N, DOUT), jnp.float32),
        grid=(B,),
        in_specs=[
            pl.BlockSpec((1, N, D), lambda b: (b, 0, 0)),
            pl.BlockSpec((D, DOUT), lambda b: (0, 0)),
        ],
        out_specs=pl.BlockSpec((1, N, DOUT), lambda b: (b, 0, 0)),
    )(x, Wout)
    return out


# speedup vs baseline: 17.5115x; 17.5115x over previous
import functools

import jax
import jax.numpy as jnp
import numpy as np
from jax.experimental import pallas as pl

B, N, DIN, D, H, DH, L, K, DFF, DOUT = 4, 2048, 2, 128, 4, 32, 3, 16, 256, 128
RB = 256
NB = N // RB
NEG = -1e30
EPS = 1e-5
BF = jnp.bfloat16
F32 = jnp.float32


def _bfc(t):
    return t.astype(BF).astype(F32)


def _mm(a, b):
    return jnp.dot(a.astype(BF), b.astype(BF), preferred_element_type=F32)


def _prep_body(pos_ref, posT_ref, h_ref, wp_ref, bp_ref, wrel2_ref,
               x_ref, idx_ref, bias_ref, pwt_ref):
    nb = pl.program_id(1)
    pos = pos_ref[...]
    posT = posT_ref[...]
    hh = h_ref[...]
    wp = wp_ref[...]
    bp = bp_ref[...]
    x_ref[...] = (_bfc(hh[:, 0:1]) * _bfc(wp[0:1, :])
                  + _bfc(hh[:, 1:2]) * _bfc(wp[1:2, :])) + bp
    w2 = wrel2_ref[...]
    posT_blk = posT_ref[:, pl.ds(nb * RB, RB)]
    pwt_ref[...] = w2[:, 0:1] * posT_blk[0:1, :] + w2[:, 1:2] * posT_blk[1:2, :]
    px_c, py_c = pos[:, 0:1], pos[:, 1:2]
    px_r, py_r = posT[0:1, :], posT[1:2, :]
    sq_c = px_c * px_c + py_c * py_c
    sq_r = px_r * px_r + py_r * py_r
    mm = _bfc(px_c) * _bfc(px_r) + _bfc(py_c) * _bfc(py_r)
    d2 = (sq_c + sq_r) - 2.0 * mm
    lane = jax.lax.broadcasted_iota(jnp.int32, (RB, N), 1)
    bias = jnp.full((RB, N), NEG, F32)
    cols = []
    for _ in range(K):
        m = jnp.min(d2, axis=-1, keepdims=True)
        a2 = jnp.min(jnp.where(d2 == m, lane, jnp.int32(N)), axis=-1,
                     keepdims=True)
        sel = lane == a2
        bias = jnp.where(sel, 0.0, bias)
        d2 = jnp.where(sel, 1e30, d2)
        cols.append(a2)
    idx_ref[...] = jnp.concatenate(cols, axis=1)
    bias_ref[...] = bias.astype(BF)


def _qkv_body(x_ref, s_ref, b_ref, wq_ref, wk_ref, wv_ref,
              q_ref, kT_ref, v_ref):
    x = x_ref[...]
    m = jnp.mean(x, -1, keepdims=True)
    xc = x - m
    var = jnp.mean(xc * xc, -1, keepdims=True)
    xn = (xc / jnp.sqrt(var + EPS)) * s_ref[...] + b_ref[...]
    q_ref[...] = _mm(xn, wq_ref[...]).astype(BF)
    kT_ref[...] = _mm(xn, wk_ref[...]).astype(BF).T
    v_ref[...] = _mm(xn, wv_ref[...]).astype(BF)


def _attn_body(q_ref, kT_ref, v_ref, bias_ref, pwt_ref, x_ref, wo_ref,
               o_ref, *, l):
    q = q_ref[...]
    bias = bias_ref[...]
    rdh = np.float32(np.sqrt(DH))
    outs = []
    for h in range(H):
        S = jnp.dot(q[:, h * DH:(h + 1) * DH], kT_ref[h * DH:(h + 1) * DH, :],
                    preferred_element_type=F32)
        logits = S / rdh + pwt_ref[l * H + h:l * H + h + 1, :] + bias.astype(F32)
        mx = jnp.max(logits, -1, keepdims=True)
        e = jnp.exp(logits - mx)
        s = jnp.sum(e, -1, keepdims=True)
        p = e / s
        outs.append(_mm(p, v_ref[:, h * DH:(h + 1) * DH]))
    o = jnp.concatenate(outs, axis=-1)
    o_ref[...] = x_ref[...] + _mm(o, wo_ref[...])


def _mlp_body(x_ref, s_ref, b_ref, w1_ref, b1_ref, w2_ref, b2_ref, *rest):
    if len(rest) == 2:
        wout_ref, out_ref = rest
    else:
        wout_ref, (out_ref,) = None, rest
    x = x_ref[...]
    m = jnp.mean(x, -1, keepdims=True)
    xc = x - m
    var = jnp.mean(xc * xc, -1, keepdims=True)
    xn = (xc / jnp.sqrt(var + EPS)) * s_ref[...] + b_ref[...]
    hmid = jax.nn.gelu(_mm(xn, w1_ref[...]) + b1_ref[...])
    y = x + (_mm(hmid, w2_ref[...]) + b2_ref[...])
    if wout_ref is not None:
        out_ref[...] = _mm(y, wout_ref[...])
    else:
        out_ref[...] = y


def _blk(b, nb):
    return (b, nb, 0)


def _prep_call(input_pos, posT, h, Wproj, bproj2, wrel2):
    grid = (B, NB)
    return pl.pallas_call(
        _prep_body,
        grid=grid,
        in_specs=[
            pl.BlockSpec((None, RB, DIN), _blk),
            pl.BlockSpec((None, DIN, N), lambda b, nb: (b, 0, 0)),
            pl.BlockSpec((None, RB, DIN), _blk),
            pl.BlockSpec((DIN, D), lambda b, nb: (0, 0)),
            pl.BlockSpec((1, D), lambda b, nb: (0, 0)),
            pl.BlockSpec((L * H, DIN), lambda b, nb: (0, 0)),
        ],
        out_specs=[
            pl.BlockSpec((None, RB, D), _blk),
            pl.BlockSpec((None, RB, K), _blk),
            pl.BlockSpec((None, RB, N), _blk),
            pl.BlockSpec((None, L * H, RB), lambda b, nb: (b, 0, nb)),
        ],
        out_shape=[
            jax.ShapeDtypeStruct((B, N, D), F32),
            jax.ShapeDtypeStruct((B, N, K), jnp.int32),
            jax.ShapeDtypeStruct((B, N, N), BF),
            jax.ShapeDtypeStruct((B, L * H, N), F32),
        ],
    )(input_pos, posT, h, Wproj, bproj2, wrel2)


def kernel(h, input_pos, Wproj, bproj, Wq, Wk, Wv, Wo, Wrel, ln1_s, ln1_b,
           W1, b1, W2, b2, ln2_s, ln2_b, Wout):
    posT = jnp.transpose(input_pos, (0, 2, 1))
    wrel2 = jnp.transpose(Wrel, (0, 2, 1)).reshape(L * H, DIN)
    bproj2 = bproj.reshape(1, D)
    grid = (B, NB)
    x, idx, bias, pwt = _prep_call(input_pos, posT, h, Wproj, bproj2, wrel2)

    out = None
    for l in range(L):
        q, kT, v = pl.pallas_call(
            _qkv_body,
            grid=grid,
            in_specs=[
                pl.BlockSpec((None, RB, D), _blk),
                pl.BlockSpec((None, 1, D), lambda b, nb, l=l: (l, 0, 0)),
                pl.BlockSpec((None, 1, D), lambda b, nb, l=l: (l, 0, 0)),
                pl.BlockSpec((None, D, D), lambda b, nb, l=l: (l, 0, 0)),
                pl.BlockSpec((None, D, D), lambda b, nb, l=l: (l, 0, 0)),
                pl.BlockSpec((None, D, D), lambda b, nb, l=l: (l, 0, 0)),
            ],
            out_specs=[
                pl.BlockSpec((None, RB, D), _blk),
                pl.BlockSpec((None, D, RB), lambda b, nb: (b, 0, nb)),
                pl.BlockSpec((None, RB, D), _blk),
            ],
            out_shape=[
                jax.ShapeDtypeStruct((B, N, D), BF),
                jax.ShapeDtypeStruct((B, D, N), BF),
                jax.ShapeDtypeStruct((B, N, D), BF),
            ],
        )(x, ln1_s.reshape(L, 1, D), ln1_b.reshape(L, 1, D), Wq, Wk, Wv)

        xo = pl.pallas_call(
            functools.partial(_attn_body, l=l),
            grid=grid,
            in_specs=[
                pl.BlockSpec((None, RB, D), _blk),
                pl.BlockSpec((None, D, N), lambda b, nb: (b, 0, 0)),
                pl.BlockSpec((None, N, D), lambda b, nb: (b, 0, 0)),
                pl.BlockSpec((None, RB, N), _blk),
                pl.BlockSpec((None, L * H, N), lambda b, nb: (b, 0, 0)),
                pl.BlockSpec((None, RB, D), _blk),
                pl.BlockSpec((None, D, D), lambda b, nb, l=l: (l, 0, 0)),
            ],
            out_specs=pl.BlockSpec((None, RB, D), _blk),
            out_shape=jax.ShapeDtypeStruct((B, N, D), F32),
        )(q, kT, v, bias, pwt, x, Wo)

        mlp_in_specs = [
            pl.BlockSpec((None, RB, D), _blk),
            pl.BlockSpec((None, 1, D), lambda b, nb, l=l: (l, 0, 0)),
            pl.BlockSpec((None, 1, D), lambda b, nb, l=l: (l, 0, 0)),
            pl.BlockSpec((None, D, DFF), lambda b, nb, l=l: (l, 0, 0)),
            pl.BlockSpec((None, 1, DFF), lambda b, nb, l=l: (l, 0, 0)),
            pl.BlockSpec((None, DFF, D), lambda b, nb, l=l: (l, 0, 0)),
            pl.BlockSpec((None, 1, D), lambda b, nb, l=l: (l, 0, 0)),
        ]
        mlp_args = [xo, ln2_s.reshape(L, 1, D), ln2_b.reshape(L, 1, D),
                    W1, b1.reshape(L, 1, DFF), W2, b2.reshape(L, 1, D)]
        if l == L - 1:
            mlp_in_specs.append(pl.BlockSpec((D, DOUT), lambda b, nb: (0, 0)))
            mlp_args.append(Wout)
            out = pl.pallas_call(
                _mlp_body,
                grid=grid,
                in_specs=mlp_in_specs,
                out_specs=pl.BlockSpec((None, RB, DOUT), _blk),
                out_shape=jax.ShapeDtypeStruct((B, N, DOUT), F32),
            )(*mlp_args)
        else:
            x = pl.pallas_call(
                _mlp_body,
                grid=grid,
                in_specs=mlp_in_specs,
                out_specs=pl.BlockSpec((None, RB, D), _blk),
                out_shape=jax.ShapeDtypeStruct((B, N, D), F32),
            )(*mlp_args)
    return out


# speedup vs baseline: 18.1253x; 1.0350x over previous
import functools

import jax
import jax.numpy as jnp
import numpy as np
from jax.experimental import pallas as pl

B, N, DIN, D, H, DH, L, K, DFF, DOUT = 4, 2048, 2, 128, 4, 32, 3, 16, 256, 128
RB = 256
NB = N // RB
NEG = -1e30
EPS = 1e-5
BF = jnp.bfloat16
F32 = jnp.float32


def _bfc(t):
    return t.astype(BF).astype(F32)


def _mm(a, b):
    return jnp.dot(a.astype(BF), b.astype(BF), preferred_element_type=F32)


def _prep_body(pos_ref, posT_ref, h_ref, wp_ref, bp_ref, wrel2_ref,
               x_ref, idx_ref, bias_ref, pwt_ref):
    nb = pl.program_id(1)
    pos = pos_ref[...]
    posT = posT_ref[...]
    hh = h_ref[...]
    wp = wp_ref[...]
    bp = bp_ref[...]
    x_ref[...] = (_bfc(hh[:, 0:1]) * _bfc(wp[0:1, :])
                  + _bfc(hh[:, 1:2]) * _bfc(wp[1:2, :])) + bp
    w2 = wrel2_ref[...]
    posT_blk = posT_ref[:, pl.ds(nb * RB, RB)]
    pwt_ref[...] = w2[:, 0:1] * posT_blk[0:1, :] + w2[:, 1:2] * posT_blk[1:2, :]
    px_c, py_c = pos[:, 0:1], pos[:, 1:2]
    px_r, py_r = posT[0:1, :], posT[1:2, :]
    sq_c = px_c * px_c + py_c * py_c
    sq_r = px_r * px_r + py_r * py_r
    mm = _bfc(px_c) * _bfc(px_r) + _bfc(py_c) * _bfc(py_r)
    d2 = (sq_c + sq_r) - 2.0 * mm
    lane = jax.lax.broadcasted_iota(jnp.int32, (RB, N), 1)
    cols = []
    for _ in range(K):
        m = jnp.min(d2, axis=-1, keepdims=True)
        a2 = jnp.min(jnp.where(d2 == m, lane, jnp.int32(N)), axis=-1,
                     keepdims=True)
        d2 = jnp.where(lane == a2, 1e30, d2)
        cols.append(a2)
    idx_ref[...] = jnp.concatenate(cols, axis=1)
    bias_ref[...] = jnp.where(d2 == 1e30, 0.0, NEG).astype(BF)


def _qkv_body(x_ref, s_ref, b_ref, wq_ref, wk_ref, wv_ref,
              q_ref, kT_ref, v_ref):
    x = x_ref[...]
    m = jnp.mean(x, -1, keepdims=True)
    xc = x - m
    var = jnp.mean(xc * xc, -1, keepdims=True)
    xn = (xc / jnp.sqrt(var + EPS)) * s_ref[...] + b_ref[...]
    q_ref[...] = _mm(xn, wq_ref[...]).astype(BF)
    kT_ref[...] = _mm(xn, wk_ref[...]).astype(BF).T
    v_ref[...] = _mm(xn, wv_ref[...]).astype(BF)


def _attn_body(q_ref, kT_ref, v_ref, bias_ref, pwt_ref, x_ref, wo_ref,
               o_ref, *, l):
    q = q_ref[...]
    bias = bias_ref[...]
    irdh = np.float32(1.0 / np.sqrt(DH))
    biasf = bias.astype(F32)
    outs = []
    for h in range(H):
        S = jnp.dot(q[:, h * DH:(h + 1) * DH], kT_ref[h * DH:(h + 1) * DH, :],
                    preferred_element_type=F32)
        logits = S * irdh + (pwt_ref[l * H + h:l * H + h + 1, :] + biasf)
        mx = jnp.max(logits, -1, keepdims=True)
        e = jnp.exp(logits - mx)
        s = jnp.sum(e, -1, keepdims=True)
        p = e * (1.0 / s)
        outs.append(_mm(p, v_ref[:, h * DH:(h + 1) * DH]))
    o = jnp.concatenate(outs, axis=-1)
    o_ref[...] = x_ref[...] + _mm(o, wo_ref[...])


def _mlp_body(x_ref, s_ref, b_ref, w1_ref, b1_ref, w2_ref, b2_ref, *rest):
    if len(rest) == 2:
        wout_ref, out_ref = rest
    else:
        wout_ref, (out_ref,) = None, rest
    x = x_ref[...]
    m = jnp.mean(x, -1, keepdims=True)
    xc = x - m
    var = jnp.mean(xc * xc, -1, keepdims=True)
    xn = (xc / jnp.sqrt(var + EPS)) * s_ref[...] + b_ref[...]
    hmid = jax.nn.gelu(_mm(xn, w1_ref[...]) + b1_ref[...])
    y = x + (_mm(hmid, w2_ref[...]) + b2_ref[...])
    if wout_ref is not None:
        out_ref[...] = _mm(y, wout_ref[...])
    else:
        out_ref[...] = y


def _blk(b, nb):
    return (b, nb, 0)


def _prep_call(input_pos, posT, h, Wproj, bproj2, wrel2):
    grid = (B, NB)
    return pl.pallas_call(
        _prep_body,
        grid=grid,
        in_specs=[
            pl.BlockSpec((None, RB, DIN), _blk),
            pl.BlockSpec((None, DIN, N), lambda b, nb: (b, 0, 0)),
            pl.BlockSpec((None, RB, DIN), _blk),
            pl.BlockSpec((DIN, D), lambda b, nb: (0, 0)),
            pl.BlockSpec((1, D), lambda b, nb: (0, 0)),
            pl.BlockSpec((L * H, DIN), lambda b, nb: (0, 0)),
        ],
        out_specs=[
            pl.BlockSpec((None, RB, D), _blk),
            pl.BlockSpec((None, RB, K), _blk),
            pl.BlockSpec((None, RB, N), _blk),
            pl.BlockSpec((None, L * H, RB), lambda b, nb: (b, 0, nb)),
        ],
        out_shape=[
            jax.ShapeDtypeStruct((B, N, D), F32),
            jax.ShapeDtypeStruct((B, N, K), jnp.int32),
            jax.ShapeDtypeStruct((B, N, N), BF),
            jax.ShapeDtypeStruct((B, L * H, N), F32),
        ],
    )(input_pos, posT, h, Wproj, bproj2, wrel2)


def kernel(h, input_pos, Wproj, bproj, Wq, Wk, Wv, Wo, Wrel, ln1_s, ln1_b,
           W1, b1, W2, b2, ln2_s, ln2_b, Wout):
    posT = jnp.transpose(input_pos, (0, 2, 1))
    wrel2 = jnp.transpose(Wrel, (0, 2, 1)).reshape(L * H, DIN)
    bproj2 = bproj.reshape(1, D)
    grid = (B, NB)
    x, idx, bias, pwt = _prep_call(input_pos, posT, h, Wproj, bproj2, wrel2)

    out = None
    for l in range(L):
        q, kT, v = pl.pallas_call(
            _qkv_body,
            grid=grid,
            in_specs=[
                pl.BlockSpec((None, RB, D), _blk),
                pl.BlockSpec((None, 1, D), lambda b, nb, l=l: (l, 0, 0)),
                pl.BlockSpec((None, 1, D), lambda b, nb, l=l: (l, 0, 0)),
                pl.BlockSpec((None, D, D), lambda b, nb, l=l: (l, 0, 0)),
                pl.BlockSpec((None, D, D), lambda b, nb, l=l: (l, 0, 0)),
                pl.BlockSpec((None, D, D), lambda b, nb, l=l: (l, 0, 0)),
            ],
            out_specs=[
                pl.BlockSpec((None, RB, D), _blk),
                pl.BlockSpec((None, D, RB), lambda b, nb: (b, 0, nb)),
                pl.BlockSpec((None, RB, D), _blk),
            ],
            out_shape=[
                jax.ShapeDtypeStruct((B, N, D), BF),
                jax.ShapeDtypeStruct((B, D, N), BF),
                jax.ShapeDtypeStruct((B, N, D), BF),
            ],
        )(x, ln1_s.reshape(L, 1, D), ln1_b.reshape(L, 1, D), Wq, Wk, Wv)

        xo = pl.pallas_call(
            functools.partial(_attn_body, l=l),
            grid=grid,
            in_specs=[
                pl.BlockSpec((None, RB, D), _blk),
                pl.BlockSpec((None, D, N), lambda b, nb: (b, 0, 0)),
                pl.BlockSpec((None, N, D), lambda b, nb: (b, 0, 0)),
                pl.BlockSpec((None, RB, N), _blk),
                pl.BlockSpec((None, L * H, N), lambda b, nb: (b, 0, 0)),
                pl.BlockSpec((None, RB, D), _blk),
                pl.BlockSpec((None, D, D), lambda b, nb, l=l: (l, 0, 0)),
            ],
            out_specs=pl.BlockSpec((None, RB, D), _blk),
            out_shape=jax.ShapeDtypeStruct((B, N, D), F32),
        )(q, kT, v, bias, pwt, x, Wo)

        mlp_in_specs = [
            pl.BlockSpec((None, RB, D), _blk),
            pl.BlockSpec((None, 1, D), lambda b, nb, l=l: (l, 0, 0)),
            pl.BlockSpec((None, 1, D), lambda b, nb, l=l: (l, 0, 0)),
            pl.BlockSpec((None, D, DFF), lambda b, nb, l=l: (l, 0, 0)),
            pl.BlockSpec((None, 1, DFF), lambda b, nb, l=l: (l, 0, 0)),
            pl.BlockSpec((None, DFF, D), lambda b, nb, l=l: (l, 0, 0)),
            pl.BlockSpec((None, 1, D), lambda b, nb, l=l: (l, 0, 0)),
        ]
        mlp_args = [xo, ln2_s.reshape(L, 1, D), ln2_b.reshape(L, 1, D),
                    W1, b1.reshape(L, 1, DFF), W2, b2.reshape(L, 1, D)]
        if l == L - 1:
            mlp_in_specs.append(pl.BlockSpec((D, DOUT), lambda b, nb: (0, 0)))
            mlp_args.append(Wout)
            out = pl.pallas_call(
                _mlp_body,
                grid=grid,
                in_specs=mlp_in_specs,
                out_specs=pl.BlockSpec((None, RB, DOUT), _blk),
                out_shape=jax.ShapeDtypeStruct((B, N, DOUT), F32),
            )(*mlp_args)
        else:
            x = pl.pallas_call(
                _mlp_body,
                grid=grid,
                in_specs=mlp_in_specs,
                out_specs=pl.BlockSpec((None, RB, D), _blk),
                out_shape=jax.ShapeDtypeStruct((B, N, D), F32),
            )(*mlp_args)
    return out
